# Initial kernel scaffold; baseline (speedup 1.0000x reference)
import jax
import jax.numpy as jnp
from jax.experimental import pallas as pl


def kernel(x, era_latlons, h_latlons, era_trainable, h_trainable, e2h_trainable, h2e_trainable, h2h_trainable, e2h_edge_attr, h2h_edge_attr, h2e_edge_attr, e2h_edge_index, h2h_edge_index, h2e_edge_index, z, params):
    raise NotImplementedError("write your pallas kernel here")


# speedup vs baseline: 1.5719x; 1.5719x over previous
import functools

import jax
import jax.numpy as jnp
from jax import lax
from jax.experimental import pallas as pl
from jax.experimental.pallas import tpu as pltpu
from jax.experimental.pallas import tpu_sc as plsc

F32 = jnp.float32
_NC, _NS, _NW = 2, 16, 32
_K = 128
_SC_PARAMS = pltpu.CompilerParams(use_tc_tiling_on_sc=False)


def _rows(body, ins, weights, out_dims, blk=512):
    n = ins[0].shape[0]
    grid = (pl.cdiv(n, blk),)
    in_specs = ([pl.BlockSpec((blk, a.shape[1]), lambda i: (i, 0)) for a in ins]
                + [pl.BlockSpec(w.shape, lambda i: (0, 0)) for w in weights])
    out_specs = [pl.BlockSpec((blk, d), lambda i: (i, 0)) for d in out_dims]
    out_shape = [jax.ShapeDtypeStruct((n, d), F32) for d in out_dims]
    ni, nw = len(ins), len(weights)

    def kfn(*refs):
        xs = [refs[k][...] for k in range(ni)]
        ws = [refs[ni + k][...] for k in range(nw)]
        outs = body(xs, ws)
        for r, o in zip(refs[ni + nw:], outs):
            r[...] = o

    return pl.pallas_call(kfn, grid=grid, in_specs=in_specs,
                          out_specs=out_specs, out_shape=out_shape)(*ins, *weights)


def _silu(x):
    return x * jax.nn.sigmoid(x)


def _ln(x, g, b, dim):
    mu = jnp.sum(x, -1, keepdims=True) * (1.0 / dim)
    if dim == x.shape[-1]:
        xc = x - mu
    else:
        mask = lax.broadcasted_iota(jnp.int32, x.shape, 1) < dim
        xc = jnp.where(mask, x - mu, 0.0)
    var = jnp.sum(xc * xc, -1, keepdims=True) * (1.0 / dim)
    return xc * lax.rsqrt(var + 1e-5) * g + b


def _mm(x, w):
    return jnp.dot(x, w, preferred_element_type=F32)


def _gather2sum(a, b, si, di):
    e = si.shape[0]
    d = a.shape[1]
    nch = e // _K
    nloop = pl.cdiv(nch, _NW)
    mesh = plsc.VectorSubcoreMesh(core_axis_name="c", subcore_axis_name="s")

    @functools.partial(
        pl.kernel,
        out_type=jax.ShapeDtypeStruct((e, d), F32),
        mesh=mesh,
        compiler_params=_SC_PARAMS,
        scratch_types=[
            pltpu.VMEM((_K,), jnp.int32),
            pltpu.VMEM((_K,), jnp.int32),
            pltpu.VMEM((_K, d), F32),
            pltpu.VMEM((_K, d), F32),
            pltpu.SemaphoreType.DMA,
        ],
    )
    def k(a_hbm, b_hbm, si_hbm, di_hbm, out_hbm, siv, div, ra, rb, sem):
        wid = lax.axis_index("s") * _NC + lax.axis_index("c")

        def step(i, carry):
            c = wid + i * _NW

            @pl.when(c < nch)
            def _():
                base = c * _K
                pltpu.sync_copy(si_hbm.at[pl.ds(base, _K)], siv)
                pltpu.sync_copy(di_hbm.at[pl.ds(base, _K)], div)
                pltpu.async_copy(a_hbm.at[siv], ra, sem).wait()
                pltpu.async_copy(b_hbm.at[div], rb, sem).wait()

                def add(r, cc):
                    for l in range(0, d, 16):
                        ra[r, pl.ds(l, 16)] = ra[r, pl.ds(l, 16)] + rb[r, pl.ds(l, 16)]
                    return cc

                lax.fori_loop(0, _K, add, 0)
                pltpu.sync_copy(ra, out_hbm.at[pl.ds(base, _K)])
            return carry

        lax.fori_loop(0, nloop, step, 0)

    return k(a, b, si, di)


def _scatter_partials(msg, di, v):
    e, d = msg.shape
    nch = e // _K
    nch_half = nch // _NC
    rows_per = v // _NS
    zr = 125
    nz = rows_per // zr
    nloop = pl.cdiv(nch_half, _NS)
    mesh = plsc.VectorSubcoreMesh(core_axis_name="c", subcore_axis_name="s")

    @functools.partial(
        pl.kernel,
        out_type=(jax.ShapeDtypeStruct((v, d), F32),
                  jax.ShapeDtypeStruct((v, d), F32)),
        mesh=mesh,
        compiler_params=_SC_PARAMS,
        scratch_types=[
            pltpu.VMEM((_K,), jnp.int32),
            pltpu.VMEM((_K, d), F32),
            pltpu.VMEM((zr, d), F32),
            pltpu.VMEM_SHARED((v, d), F32),
        ],
    )
    def k(msg_hbm, di_hbm, out0, out1, div, mbuf, zbuf, accum):
        cid = lax.axis_index("c")
        sid = lax.axis_index("s")

        def zrow(r, cc):
            for l in range(0, d, 16):
                zbuf[r, pl.ds(l, 16)] = jnp.zeros((16,), F32)
            return cc

        lax.fori_loop(0, zr, zrow, 0)
        for j in range(nz):
            pltpu.sync_copy(zbuf, accum.at[pl.ds(sid * rows_per + j * zr, zr)])
        plsc.subcore_barrier()

        def step(i, carry):
            c = sid + i * _NS

            @pl.when(c < nch_half)
            def _():
                base = (cid * nch_half + c) * _K
                pltpu.sync_copy(di_hbm.at[pl.ds(base, _K)], div)
                pltpu.sync_copy(msg_hbm.at[pl.ds(base, _K)], mbuf)
                pltpu.sync_copy(mbuf, accum.at[div], add=True)
            return carry

        lax.fori_loop(0, nloop, step, 0)
        plsc.subcore_barrier()

        r0 = sid * rows_per

        @pl.when(cid == 0)
        def _():
            pltpu.sync_copy(accum.at[pl.ds(r0, rows_per)], out0.at[pl.ds(r0, rows_per)])

        @pl.when(cid == 1)
        def _():
            pltpu.sync_copy(accum.at[pl.ds(r0, rows_per)], out1.at[pl.ds(r0, rows_per)])

    return k(msg, di)


def _scatter_featsplit(msg, di, v, fc=32):
    e, d = msg.shape
    nch = e // _K
    rows_per = v // _NS
    zr = 125
    nz = rows_per // zr
    nloop = pl.cdiv(nch, _NS)
    n_fc = d // fc // _NC
    mesh = plsc.VectorSubcoreMesh(core_axis_name="c", subcore_axis_name="s")

    @functools.partial(
        pl.kernel,
        out_type=jax.ShapeDtypeStruct((v, d), F32),
        mesh=mesh,
        compiler_params=_SC_PARAMS,
        scratch_types=[
            pltpu.VMEM((_K,), jnp.int32),
            pltpu.VMEM((_K, fc), F32),
            pltpu.VMEM((zr, fc), F32),
            pltpu.VMEM_SHARED((v, fc), F32),
        ],
    )
    def k(msg_hbm, di_hbm, out, div, mbuf, zbuf, accum):
        cid = lax.axis_index("c")
        sid = lax.axis_index("s")

        def zrow(r, cc):
            for l in range(0, fc, 16):
                zbuf[r, pl.ds(l, 16)] = jnp.zeros((16,), F32)
            return cc

        lax.fori_loop(0, zr, zrow, 0)
        r0 = sid * rows_per

        for cidv in range(_NC):
            @pl.when(cid == cidv)
            def _(cidv=cidv):
                for j in range(n_fc):
                    f0 = (cidv * n_fc + j) * fc
                    for z in range(nz):
                        pltpu.sync_copy(zbuf, accum.at[pl.ds(r0 + z * zr, zr)])
                    plsc.subcore_barrier()

                    def step(i, carry, f0=f0):
                        c = sid + i * _NS

                        @pl.when(c < nch)
                        def _():
                            base = c * _K
                            pltpu.sync_copy(di_hbm.at[pl.ds(base, _K)], div)
                            pltpu.sync_copy(
                                msg_hbm.at[pl.ds(base, _K), pl.ds(f0, fc)], mbuf)
                            pltpu.sync_copy(mbuf, accum.at[div], add=True)
                        return carry

                    lax.fori_loop(0, nloop, step, 0)
                    plsc.subcore_barrier()
                    pltpu.sync_copy(accum.at[pl.ds(r0, rows_per)],
                                    out.at[pl.ds(r0, rows_per), pl.ds(f0, fc)])
                    plsc.subcore_barrier()

    return k(msg, di)


def _pad2(w, r, c):
    out = jnp.zeros((r, c), F32)
    return out.at[: w.shape[0], : w.shape[1]].set(w)


def _row(v, c=None):
    v = v.reshape(1, -1)
    if c is not None and v.shape[1] != c:
        v = _pad2(v, 1, c)
    return v


def _mlp_parts(m):
    (w0, b0), (w1, b1) = m["layers"]
    g, be = m["ln"]
    return w0, b0, w1, b1, g, be


def _emb_proj(x, mlp, projs, dim):
    w0, b0, w1, b1, g, be = _mlp_parts(mlp)
    ws = [w0, _row(b0), w1, _row(b1), _row(g), _row(be)] + list(projs)

    def body(xs, ws):
        (xx,) = xs
        w0_, b0_, w1_, b1_, g_, be_ = ws[:6]
        h = _silu(_mm(xx, w0_) + b0_)
        y = _ln(_mm(h, w1_) + b1_, g_, be_, dim)
        return (y,) + tuple(_mm(y, p) for p in ws[6:])

    return _rows(body, [x], ws, [ws[0].shape[1]] * 0 + [w1.shape[1]] + [p.shape[1] for p in projs])


def _msg_stage(attrs, s, emb_mlp, we_e, be0, we1, be1, g, be, dim, pad, with_ee_out=False):
    w0, b0, w1, b1, ge, bee = _mlp_parts(emb_mlp)
    ws = [_pad2(w0, w0.shape[0], pad), _row(b0, pad), _pad2(w1, pad, pad),
          _row(b1, pad), _row(ge, pad), _row(bee, pad),
          _pad2(we_e, pad, pad), _row(be0, pad), _pad2(we1, pad, pad),
          _row(be1, pad), _row(g, pad), _row(be, pad)]

    def body(xs, ws):
        a, sg = xs
        (w0_, b0_, w1_, b1_, ge_, bee_, wee_, be0_, we1_, be1_, g_, be_) = ws
        ee = _ln(_mm(_silu(_mm(a, w0_) + b0_), w1_) + b1_, ge_, bee_, dim)
        cc = _mm(ee, wee_) + be0_
        h1 = _silu(sg + cc)
        msg = _ln(_mm(h1, we1_) + be1_, g_, be_, dim)
        if with_ee_out:
            return msg, ee + msg
        return (msg,)

    outs = _rows(body, [attrs, s], ws, [pad, pad] if with_ee_out else [pad])
    return outs


def _msg_stage2(ep, s, we_e, be0, we1, be1, g, be, dim, pad):
    ws = [_pad2(we_e, pad, pad), _row(be0, pad), _pad2(we1, pad, pad),
          _row(be1, pad), _row(g, pad), _row(be, pad)]

    def body(xs, ws):
        epb, sg = xs
        wee_, be0_, we1_, be1_, g_, be_ = ws
        cc = _mm(epb, wee_) + be0_
        h1 = _silu(sg + cc)
        msg = _ln(_mm(h1, we1_) + be1_, g_, be_, dim)
        return (msg,)

    (msg,) = _rows(body, [ep, s], ws, [pad])
    return msg


def _node_update(xn, aggs, node_mlp, dim, pad, projs=()):
    w0, b0, w1, b1, g, be = _mlp_parts(node_mlp)
    dtrue = dim
    wa = _pad2(w0[:dtrue], pad, pad)
    wb = _pad2(w0[dtrue:], pad, pad)
    ws = [wa, wb, _row(b0, pad), _pad2(w1, pad, pad), _row(b1, pad),
          _row(g, pad), _row(be, pad)] + list(projs)

    def body(xs, ws):
        xx = xs[0]
        agg = xs[1]
        for extra in xs[2:]:
            agg = agg + extra
        wa_, wb_, b0_, w1_, b1_, g_, be_ = ws[:7]
        h = _silu(_mm(xx, wa_) + _mm(agg, wb_) + b0_)
        y = xx + _ln(_mm(h, w1_) + b1_, g_, be_, dim)
        return (y,) + tuple(_mm(y, p) for p in ws[7:])

    return _rows(body, [xn] + list(aggs), ws, [pad] + [p.shape[1] for p in projs])


def kernel(x, era_latlons, h_latlons, era_trainable, h_trainable,
           e2h_trainable, h2e_trainable, h2h_trainable, e2h_edge_attr,
           h2h_edge_attr, h2e_edge_attr, e2h_edge_index, h2h_edge_index,
           h2e_edge_index, z, params):
    era, h = era_latlons.shape[0], h_latlons.shape[0]
    hid = 128
    p = 144

    bs, ens, ms, n, f = x.shape
    xf = jnp.transpose(x, (0, 1, 3, 2, 4)).reshape(n, ms * f)
    x_era = jnp.concatenate([xf, era_latlons, era_trainable], axis=1)
    x_h = jnp.concatenate([h_latlons, h_trainable], axis=1)
    e2h_a = jnp.concatenate([e2h_edge_attr, e2h_trainable], axis=1)
    h2h_a = jnp.concatenate([h2h_edge_attr, h2h_trainable], axis=1)
    h2e_a = jnp.concatenate([h2e_edge_attr, h2e_trainable], axis=1)
    e2h_s = e2h_edge_index[0].astype(jnp.int32)
    e2h_d = e2h_edge_index[1].astype(jnp.int32)
    h2h_s = h2h_edge_index[0].astype(jnp.int32)
    h2h_d = h2h_edge_index[1].astype(jnp.int32)
    h2e_s = h2e_edge_index[0].astype(jnp.int32)
    h2e_d = h2e_edge_index[1].astype(jnp.int32)

    fm = params["fmap"]
    pr = params["proc"]
    bm = params["bmap"]

    fblk = fm["blocks"][0]
    fe_w0, fe_b0, fe_w1, fe_b1, fe_g, fe_be = _mlp_parts(fblk["edge"])
    xs, a_e = _emb_proj(x_era, fm["src_emb"], [fe_w0[:hid]], hid)
    xd, b_e = _emb_proj(x_h, fm["dst_emb"], [fe_w0[hid:2 * hid]], hid)
    s_e = _gather2sum(a_e, b_e, e2h_s, e2h_d)
    (msg_e,) = _msg_stage(e2h_a, s_e, fm["edge_emb"], fe_w0[2 * hid:], fe_b0,
                          fe_w1, fe_b1, fe_g, fe_be, hid, hid)
    agg0, agg1 = _scatter_partials(msg_e, e2h_d, h)
    (x_latent,) = _node_update(xd, [agg0, agg1], fblk["node"], hid, hid)

    d129 = hid + 1
    xp = jnp.concatenate(
        [x_latent, z, jnp.zeros((h, p - d129), F32)], axis=1)

    b1k, b2k = pr["blocks"]
    p1_w0, p1_b0, p1_w1, p1_b1, p1_g, p1_be = _mlp_parts(b1k["edge"])
    p2_w0, p2_b0, p2_w1, p2_b1, p2_g, p2_be = _mlp_parts(b2k["edge"])

    def _ab_body(xs_, ws_):
        (xx,) = xs_
        return _mm(xx, ws_[0]), _mm(xx, ws_[1])

    a1, b1 = _rows(_ab_body, [xp],
                   [_pad2(p1_w0[:d129], p, p), _pad2(p1_w0[d129:2 * d129], p, p)],
                   [p, p])

    s1 = _gather2sum(a1, b1, h2h_s, h2h_d)
    msg1, ep2 = _msg_stage(h2h_a, s1, pr["edge_emb"], p1_w0[2 * d129:], p1_b0,
                           p1_w1, p1_b1, p1_g, p1_be, d129, p, with_ee_out=True)
    pagg0, pagg1 = _scatter_partials(msg1, h2h_d, h)
    xp2, a2, b2 = _node_update(
        xp, [pagg0, pagg1], b1k["node"], d129, p,
        projs=[_pad2(p2_w0[:d129], p, p), _pad2(p2_w0[d129:2 * d129], p, p)])

    s2 = _gather2sum(a2, b2, h2h_s, h2h_d)
    msg2 = _msg_stage2(ep2, s2, p2_w0[2 * d129:], p2_b0, p2_w1, p2_b1,
                       p2_g, p2_be, d129, p)
    pagg0b, pagg1b = _scatter_partials(msg2, h2h_d, h)

    n2_w0, n2_b0, n2_w1, n2_b1, n2_g, n2_be = _mlp_parts(b2k["node"])

    def _node2_body(xs_, ws_):
        xx, g0, g1, xl = xs_
        agg = g0 + g1
        wa_, wb_, b0_, w1_, b1_, g_, be_ = ws_
        hh = _silu(_mm(xx, wa_) + _mm(agg, wb_) + b0_)
        y = xx + _ln(_mm(hh, w1_) + b1_, g_, be_, d129)
        return (y[:, :hid] + xl,)

    (xlp,) = _rows(_node2_body, [xp2, pagg0b, pagg1b, x_latent],
                   [_pad2(n2_w0[:d129], p, p), _pad2(n2_w0[d129:2 * d129], p, p),
                    _row(n2_b0, p), _pad2(n2_w1, p, p), _row(n2_b1, p),
                    _row(n2_g, p), _row(n2_be, p)],
                   [hid])

    dblk = bm["blocks"][0]
    de_w0, de_b0, de_w1, de_b1, de_g, de_be = _mlp_parts(dblk["edge"])

    sw0, sb0, sw1, sb1, sg, sbe = _mlp_parts(bm["src_emb"])

    def _src_body(xs_, ws_):
        (xx,) = xs_
        w0_, b0_, w1_, b1_, g_, be_, pa_ = ws_
        y = _ln(_mm(_silu(_mm(xx, w0_) + b0_), w1_) + b1_, g_, be_, hid)
        return (_mm(y, pa_),)

    (a3,) = _rows(_src_body, [xlp],
                  [sw0, _row(sb0), sw1, _row(sb1), _row(sg), _row(sbe),
                   de_w0[:hid]], [hid])

    xd2, b3 = _emb_proj(xs, bm["dst_emb"], [de_w0[hid:2 * hid]], hid)

    s3 = _gather2sum(a3, b3, h2e_s, h2e_d)
    (msg3,) = _msg_stage(h2e_a, s3, bm["edge_emb"], de_w0[2 * hid:], de_b0,
                         de_w1, de_b1, de_g, de_be, hid, hid)
    agg3 = _scatter_featsplit(msg3, h2e_d, era)

    dn_w0, dn_b0, dn_w1, dn_b1, dn_g, dn_be = _mlp_parts(dblk["node"])
    (wo0, bo0), (wo1, bo1) = bm["out"]["layers"]
    inf = wo1.shape[1]

    def _out_body(xs_, ws_):
        xx, agg = xs_
        wa_, wb_, b0_, w1_, b1_, g_, be_, wo0_, bo0_, wo1_, bo1_ = ws_
        hh = _silu(_mm(xx, wa_) + _mm(agg, wb_) + b0_)
        y = xx + _ln(_mm(hh, w1_) + b1_, g_, be_, hid)
        ho = _silu(_mm(y, wo0_) + bo0_)
        return (_mm(ho, wo1_) + bo1_,)

    (out,) = _rows(_out_body, [xd2, agg3],
                   [dn_w0[:hid], dn_w0[hid:], _row(dn_b0), dn_w1, _row(dn_b1),
                    _row(dn_g), _row(dn_be), wo0, _row(bo0), wo1, _row(bo1)],
                   [inf])

    return out.reshape(bs, ens, n, inf)


# speedup vs baseline: 1.8838x; 1.1984x over previous
import functools

import jax
import jax.numpy as jnp
from jax import lax
from jax.experimental import pallas as pl
from jax.experimental.pallas import tpu as pltpu
from jax.experimental.pallas import tpu_sc as plsc

F32 = jnp.float32
_NC, _NS, _NW = 2, 16, 32
_K = 128
_SC_PARAMS = pltpu.CompilerParams(use_tc_tiling_on_sc=False)


def _rows(body, ins, weights, out_dims, blk=512):
    n = ins[0].shape[0]
    grid = (pl.cdiv(n, blk),)
    in_specs = ([pl.BlockSpec((blk, a.shape[1]), lambda i: (i, 0)) for a in ins]
                + [pl.BlockSpec(w.shape, lambda i: (0, 0)) for w in weights])
    out_specs = [pl.BlockSpec((blk, d), lambda i: (i, 0)) for d in out_dims]
    out_shape = [jax.ShapeDtypeStruct((n, d), F32) for d in out_dims]
    ni, nw = len(ins), len(weights)

    def kfn(*refs):
        xs = [refs[k][...] for k in range(ni)]
        ws = [refs[ni + k][...] for k in range(nw)]
        outs = body(xs, ws)
        for r, o in zip(refs[ni + nw:], outs):
            r[...] = o

    return pl.pallas_call(kfn, grid=grid, in_specs=in_specs,
                          out_specs=out_specs, out_shape=out_shape)(*ins, *weights)


def _silu(x):
    return x * jax.nn.sigmoid(x)


def _ln(x, g, b, dim):
    mu = jnp.sum(x, -1, keepdims=True) * (1.0 / dim)
    if dim == x.shape[-1]:
        xc = x - mu
    else:
        mask = lax.broadcasted_iota(jnp.int32, x.shape, 1) < dim
        xc = jnp.where(mask, x - mu, 0.0)
    var = jnp.sum(xc * xc, -1, keepdims=True) * (1.0 / dim)
    return xc * lax.rsqrt(var + 1e-5) * g + b


def _mm(x, w):
    return jnp.dot(x, w, preferred_element_type=F32)


def _gather2sum(a, b, si2, di2):
    nch, kk = si2.shape
    e = nch * kk
    d = a.shape[1]
    q = pl.cdiv(nch, _NW)
    nloop = pl.cdiv(q, 2)
    mesh = plsc.VectorSubcoreMesh(core_axis_name="c", subcore_axis_name="s")

    @functools.partial(
        pl.kernel,
        out_type=jax.ShapeDtypeStruct((e, d), F32),
        mesh=mesh,
        compiler_params=_SC_PARAMS,
        scratch_types=[
            pltpu.VMEM((q, kk), jnp.int32),
            pltpu.VMEM((q, kk), jnp.int32),
            pltpu.VMEM((kk, d), F32),
            pltpu.VMEM((kk, d), F32),
            pltpu.VMEM((kk, d), F32),
            pltpu.VMEM((kk, d), F32),
        ] + [pltpu.SemaphoreType.DMA] * 6,
    )
    def k(a_hbm, b_hbm, si_hbm, di_hbm, out_hbm, siv, div,
          ra0, rb0, ra1, rb1, sa0, sb0, sa1, sb1, sw0, sw1):
        wid = lax.axis_index("s") * _NC + lax.axis_index("c")
        first = wid * q
        n_my = lax.min(lax.max(nch - first, 0), q)
        load0 = lax.min(first, nch - q)
        delta = first - load0

        pltpu.sync_copy(si_hbm.at[pl.ds(load0, q)], siv)
        pltpu.sync_copy(di_hbm.at[pl.ds(load0, q)], div)

        ras, rbs = (ra0, ra1), (rb0, rb1)
        sas, sbs, sws = (sa0, sa1), (sb0, sb1), (sw0, sw1)

        def g_a(j, s):
            return pltpu.make_async_copy(a_hbm.at[siv.at[delta + j]], ras[s], sas[s])

        def g_b(j, s):
            return pltpu.make_async_copy(b_hbm.at[div.at[delta + j]], rbs[s], sbs[s])

        def wr(j, s):
            return pltpu.make_async_copy(
                ras[s], out_hbm.at[pl.ds((first + j) * kk, kk)], sws[s])

        def issue(j, s):
            g_a(j, s).start()
            g_b(j, s).start()

        def add(s):
            ra, rb = ras[s], rbs[s]

            def rowadd(r, cc):
                for l in range(0, d, 16):
                    ra[r, pl.ds(l, 16)] = ra[r, pl.ds(l, 16)] + rb[r, pl.ds(l, 16)]
                return cc

            lax.fori_loop(0, kk, rowadd, 0)

        @pl.when(n_my > 0)
        def _():
            issue(0, 0)

        def block(j, s):
            o = 1 - s

            @pl.when(j < n_my)
            def _():
                @pl.when(j + 1 < n_my)
                def _():
                    @pl.when(j >= 1)
                    def _():
                        wr(j - 1, o).wait()

                    issue(j + 1, o)

                g_a(j, s).wait()
                g_b(j, s).wait()
                add(s)
                wr(j, s).start()

        def step(jj, carry):
            block(2 * jj, 0)
            block(2 * jj + 1, 1)
            return carry

        lax.fori_loop(0, nloop, step, 0)

        par = lax.rem(n_my, 2)

        @pl.when(n_my >= 2)
        def _():
            @pl.when(par == 0)
            def _():
                wr(n_my - 2, 0).wait()

            @pl.when(par == 1)
            def _():
                wr(n_my - 2, 1).wait()

        @pl.when(n_my >= 1)
        def _():
            @pl.when(par == 1)
            def _():
                wr(n_my - 1, 0).wait()

            @pl.when(par == 0)
            def _():
                wr(n_my - 1, 1).wait()

    return k(a, b, si2, di2)


def _zero_shared(zbuf, accum, zr, d, r0, nz):
    def zrow(r, cc):
        for l in range(0, d, 16):
            zbuf[r, pl.ds(l, 16)] = jnp.zeros((16,), F32)
        return cc

    lax.fori_loop(0, zr, zrow, 0)
    for j in range(nz):
        pltpu.sync_copy(zbuf, accum.at[pl.ds(r0 + j * zr, zr)])


def _scatter_pipeline(msg_slice, di, accum, mbs, sms, sss, first, n_my, delta,
                      nloop, kk):
    def ld(j, s):
        return pltpu.make_async_copy(msg_slice((first + j) * kk), mbs[s], sms[s])

    def sc(j, s):
        return pltpu.make_async_copy(mbs[s], accum.at[di.at[delta + j]], sss[s])

    @pl.when(n_my > 0)
    def _():
        ld(0, 0).start()

    def block(j, s):
        o = 1 - s

        @pl.when(j < n_my)
        def _():
            @pl.when(j + 1 < n_my)
            def _():
                @pl.when(j >= 1)
                def _():
                    sc(j - 1, o).wait()

                ld(j + 1, o).start()

            ld(j, s).wait()
            sc(j, s).start(add=True)

    def step(jj, carry):
        block(2 * jj, 0)
        block(2 * jj + 1, 1)
        return carry

    lax.fori_loop(0, nloop, step, 0)

    par = lax.rem(n_my, 2)

    @pl.when(n_my >= 2)
    def _():
        @pl.when(par == 0)
        def _():
            sc(n_my - 2, 0).wait()

        @pl.when(par == 1)
        def _():
            sc(n_my - 2, 1).wait()

    @pl.when(n_my >= 1)
    def _():
        @pl.when(par == 1)
        def _():
            sc(n_my - 1, 0).wait()

        @pl.when(par == 0)
        def _():
            sc(n_my - 1, 1).wait()


def _scatter_partials(msg, di2, v):
    nch, kk = di2.shape
    d = msg.shape[1]
    nch_half = nch // _NC
    q = pl.cdiv(nch_half, _NS)
    nloop = pl.cdiv(q, 2)
    rows_per = v // _NS
    zr = 25
    nz = rows_per // zr
    mesh = plsc.VectorSubcoreMesh(core_axis_name="c", subcore_axis_name="s")

    @functools.partial(
        pl.kernel,
        out_type=(jax.ShapeDtypeStruct((v, d), F32),
                  jax.ShapeDtypeStruct((v, d), F32)),
        mesh=mesh,
        compiler_params=_SC_PARAMS,
        scratch_types=[
            pltpu.VMEM((q, kk), jnp.int32),
            pltpu.VMEM((kk, d), F32),
            pltpu.VMEM((kk, d), F32),
            pltpu.VMEM((zr, d), F32),
            pltpu.VMEM_SHARED((v, d), F32),
        ] + [pltpu.SemaphoreType.DMA] * 4,
    )
    def k(msg_hbm, di_hbm, out0, out1, div, mb0, mb1, zbuf, accum,
          sm0, sm1, ss0, ss1):
        cid = lax.axis_index("c")
        sid = lax.axis_index("s")
        r0 = sid * rows_per
        _zero_shared(zbuf, accum, zr, d, r0, nz)
        plsc.subcore_barrier()

        loc = sid * q
        n_my = lax.min(lax.max(nch_half - loc, 0), q)
        lload = lax.min(loc, nch_half - q)
        first = cid * nch_half + loc
        load0 = cid * nch_half + lload
        delta = loc - lload
        pltpu.sync_copy(di_hbm.at[pl.ds(load0, q)], div)

        _scatter_pipeline(lambda b: msg_hbm.at[pl.ds(b, kk)], div, accum,
                          (mb0, mb1), (sm0, sm1), (ss0, ss1),
                          first, n_my, delta, nloop, kk)
        plsc.subcore_barrier()

        @pl.when(cid == 0)
        def _():
            pltpu.sync_copy(accum.at[pl.ds(r0, rows_per)], out0.at[pl.ds(r0, rows_per)])

        @pl.when(cid == 1)
        def _():
            pltpu.sync_copy(accum.at[pl.ds(r0, rows_per)], out1.at[pl.ds(r0, rows_per)])

    return k(msg, di2)


def _scatter_featsplit(msg, di2, v, fc=32):
    nch, kk = di2.shape
    d = msg.shape[1]
    q = pl.cdiv(nch, _NS)
    nloop = pl.cdiv(q, 2)
    rows_per = v // _NS
    zr = 125
    nz = rows_per // zr
    n_fc = d // fc // _NC
    mesh = plsc.VectorSubcoreMesh(core_axis_name="c", subcore_axis_name="s")

    @functools.partial(
        pl.kernel,
        out_type=jax.ShapeDtypeStruct((v, d), F32),
        mesh=mesh,
        compiler_params=_SC_PARAMS,
        scratch_types=[
            pltpu.VMEM((q, kk), jnp.int32),
            pltpu.VMEM((kk, fc), F32),
            pltpu.VMEM((kk, fc), F32),
            pltpu.VMEM((zr, fc), F32),
            pltpu.VMEM_SHARED((v, fc), F32),
        ] + [pltpu.SemaphoreType.DMA] * 4,
    )
    def k(msg_hbm, di_hbm, out, div, mb0, mb1, zbuf, accum, sm0, sm1, ss0, ss1):
        cid = lax.axis_index("c")
        sid = lax.axis_index("s")
        r0 = sid * rows_per

        first = sid * q
        n_my = lax.min(lax.max(nch - first, 0), q)
        load0 = lax.min(first, nch - q)
        delta = first - load0
        pltpu.sync_copy(di_hbm.at[pl.ds(load0, q)], div)

        for cidv in range(_NC):
            @pl.when(cid == cidv)
            def _(cidv=cidv):
                for j in range(n_fc):
                    f0 = (cidv * n_fc + j) * fc
                    _zero_shared(zbuf, accum, zr, fc, r0, nz)
                    plsc.subcore_barrier()
                    _scatter_pipeline(
                        lambda b, f0=f0: msg_hbm.at[pl.ds(b, kk), pl.ds(f0, fc)],
                        div, accum, (mb0, mb1), (sm0, sm1), (ss0, ss1),
                        first, n_my, delta, nloop, kk)
                    plsc.subcore_barrier()
                    pltpu.sync_copy(accum.at[pl.ds(r0, rows_per)],
                                    out.at[pl.ds(r0, rows_per), pl.ds(f0, fc)])
                    plsc.subcore_barrier()

    return k(msg, di2)


def _pad2(w, r, c):
    out = jnp.zeros((r, c), F32)
    return out.at[: w.shape[0], : w.shape[1]].set(w)


def _row(v, c=None):
    v = v.reshape(1, -1)
    if c is not None and v.shape[1] != c:
        v = _pad2(v, 1, c)
    return v


def _mlp_parts(m):
    (w0, b0), (w1, b1) = m["layers"]
    g, be = m["ln"]
    return w0, b0, w1, b1, g, be


def _emb_proj(x, mlp, projs, dim):
    w0, b0, w1, b1, g, be = _mlp_parts(mlp)
    ws = [w0, _row(b0), w1, _row(b1), _row(g), _row(be)] + list(projs)

    def body(xs, ws):
        (xx,) = xs
        w0_, b0_, w1_, b1_, g_, be_ = ws[:6]
        h = _silu(_mm(xx, w0_) + b0_)
        y = _ln(_mm(h, w1_) + b1_, g_, be_, dim)
        return (y,) + tuple(_mm(y, p) for p in ws[6:])

    return _rows(body, [x], ws, [ws[0].shape[1]] * 0 + [w1.shape[1]] + [p.shape[1] for p in projs])


def _msg_stage(attrs, s, emb_mlp, we_e, be0, we1, be1, g, be, dim, pad, with_ee_out=False):
    w0, b0, w1, b1, ge, bee = _mlp_parts(emb_mlp)
    ws = [_pad2(w0, w0.shape[0], pad), _row(b0, pad), _pad2(w1, pad, pad),
          _row(b1, pad), _row(ge, pad), _row(bee, pad),
          _pad2(we_e, pad, pad), _row(be0, pad), _pad2(we1, pad, pad),
          _row(be1, pad), _row(g, pad), _row(be, pad)]

    def body(xs, ws):
        a, sg = xs
        (w0_, b0_, w1_, b1_, ge_, bee_, wee_, be0_, we1_, be1_, g_, be_) = ws
        ee = _ln(_mm(_silu(_mm(a, w0_) + b0_), w1_) + b1_, ge_, bee_, dim)
        cc = _mm(ee, wee_) + be0_
        h1 = _silu(sg + cc)
        msg = _ln(_mm(h1, we1_) + be1_, g_, be_, dim)
        if with_ee_out:
            return msg, ee + msg
        return (msg,)

    outs = _rows(body, [attrs, s], ws, [pad, pad] if with_ee_out else [pad])
    return outs


def _msg_stage2(ep, s, we_e, be0, we1, be1, g, be, dim, pad):
    ws = [_pad2(we_e, pad, pad), _row(be0, pad), _pad2(we1, pad, pad),
          _row(be1, pad), _row(g, pad), _row(be, pad)]

    def body(xs, ws):
        epb, sg = xs
        wee_, be0_, we1_, be1_, g_, be_ = ws
        cc = _mm(epb, wee_) + be0_
        h1 = _silu(sg + cc)
        msg = _ln(_mm(h1, we1_) + be1_, g_, be_, dim)
        return (msg,)

    (msg,) = _rows(body, [ep, s], ws, [pad])
    return msg


def _node_update(xn, aggs, node_mlp, dim, pad, projs=()):
    w0, b0, w1, b1, g, be = _mlp_parts(node_mlp)
    dtrue = dim
    wa = _pad2(w0[:dtrue], pad, pad)
    wb = _pad2(w0[dtrue:], pad, pad)
    ws = [wa, wb, _row(b0, pad), _pad2(w1, pad, pad), _row(b1, pad),
          _row(g, pad), _row(be, pad)] + list(projs)

    def body(xs, ws):
        xx = xs[0]
        agg = xs[1]
        for extra in xs[2:]:
            agg = agg + extra
        wa_, wb_, b0_, w1_, b1_, g_, be_ = ws[:7]
        h = _silu(_mm(xx, wa_) + _mm(agg, wb_) + b0_)
        y = xx + _ln(_mm(h, w1_) + b1_, g_, be_, dim)
        return (y,) + tuple(_mm(y, p) for p in ws[7:])

    return _rows(body, [xn] + list(aggs), ws, [pad] + [p.shape[1] for p in projs])


def kernel(x, era_latlons, h_latlons, era_trainable, h_trainable,
           e2h_trainable, h2e_trainable, h2h_trainable, e2h_edge_attr,
           h2h_edge_attr, h2e_edge_attr, e2h_edge_index, h2h_edge_index,
           h2e_edge_index, z, params):
    era, h = era_latlons.shape[0], h_latlons.shape[0]
    hid = 128
    p = 144

    bs, ens, ms, n, f = x.shape
    xf = jnp.transpose(x, (0, 1, 3, 2, 4)).reshape(n, ms * f)
    x_era = jnp.concatenate([xf, era_latlons, era_trainable], axis=1)
    x_h = jnp.concatenate([h_latlons, h_trainable], axis=1)
    e2h_a = jnp.concatenate([e2h_edge_attr, e2h_trainable], axis=1)
    h2h_a = jnp.concatenate([h2h_edge_attr, h2h_trainable], axis=1)
    h2e_a = jnp.concatenate([h2e_edge_attr, h2e_trainable], axis=1)
    e2h_s = e2h_edge_index[0].astype(jnp.int32).reshape(-1, _K)
    e2h_d = e2h_edge_index[1].astype(jnp.int32).reshape(-1, _K)
    h2h_s = h2h_edge_index[0].astype(jnp.int32).reshape(-1, _K)
    h2h_d = h2h_edge_index[1].astype(jnp.int32).reshape(-1, _K)
    h2h_d64 = h2h_d.reshape(-1, 64)
    h2e_s = h2e_edge_index[0].astype(jnp.int32).reshape(-1, _K)
    h2e_d = h2e_edge_index[1].astype(jnp.int32).reshape(-1, _K)

    fm = params["fmap"]
    pr = params["proc"]
    bm = params["bmap"]

    fblk = fm["blocks"][0]
    fe_w0, fe_b0, fe_w1, fe_b1, fe_g, fe_be = _mlp_parts(fblk["edge"])
    xs, a_e = _emb_proj(x_era, fm["src_emb"], [fe_w0[:hid]], hid)
    xd, b_e = _emb_proj(x_h, fm["dst_emb"], [fe_w0[hid:2 * hid]], hid)
    s_e = _gather2sum(a_e, b_e, e2h_s, e2h_d)
    (msg_e,) = _msg_stage(e2h_a, s_e, fm["edge_emb"], fe_w0[2 * hid:], fe_b0,
                          fe_w1, fe_b1, fe_g, fe_be, hid, hid)
    agg0, agg1 = _scatter_partials(msg_e, e2h_d, h)
    (x_latent,) = _node_update(xd, [agg0, agg1], fblk["node"], hid, hid)

    d129 = hid + 1
    xp = jnp.concatenate(
        [x_latent, z, jnp.zeros((h, p - d129), F32)], axis=1)

    b1k, b2k = pr["blocks"]
    p1_w0, p1_b0, p1_w1, p1_b1, p1_g, p1_be = _mlp_parts(b1k["edge"])
    p2_w0, p2_b0, p2_w1, p2_b1, p2_g, p2_be = _mlp_parts(b2k["edge"])

    def _ab_body(xs_, ws_):
        (xx,) = xs_
        return _mm(xx, ws_[0]), _mm(xx, ws_[1])

    a1, b1 = _rows(_ab_body, [xp],
                   [_pad2(p1_w0[:d129], p, p), _pad2(p1_w0[d129:2 * d129], p, p)],
                   [p, p])

    s1 = _gather2sum(a1, b1, h2h_s, h2h_d)
    msg1, ep2 = _msg_stage(h2h_a, s1, pr["edge_emb"], p1_w0[2 * d129:], p1_b0,
                           p1_w1, p1_b1, p1_g, p1_be, d129, p, with_ee_out=True)
    pagg0, pagg1 = _scatter_partials(msg1, h2h_d64, h)
    xp2, a2, b2 = _node_update(
        xp, [pagg0, pagg1], b1k["node"], d129, p,
        projs=[_pad2(p2_w0[:d129], p, p), _pad2(p2_w0[d129:2 * d129], p, p)])

    s2 = _gather2sum(a2, b2, h2h_s, h2h_d)
    msg2 = _msg_stage2(ep2, s2, p2_w0[2 * d129:], p2_b0, p2_w1, p2_b1,
                       p2_g, p2_be, d129, p)
    pagg0b, pagg1b = _scatter_partials(msg2, h2h_d64, h)

    n2_w0, n2_b0, n2_w1, n2_b1, n2_g, n2_be = _mlp_parts(b2k["node"])

    def _node2_body(xs_, ws_):
        xx, g0, g1, xl = xs_
        agg = g0 + g1
        wa_, wb_, b0_, w1_, b1_, g_, be_ = ws_
        hh = _silu(_mm(xx, wa_) + _mm(agg, wb_) + b0_)
        y = xx + _ln(_mm(hh, w1_) + b1_, g_, be_, d129)
        return (y[:, :hid] + xl,)

    (xlp,) = _rows(_node2_body, [xp2, pagg0b, pagg1b, x_latent],
                   [_pad2(n2_w0[:d129], p, p), _pad2(n2_w0[d129:2 * d129], p, p),
                    _row(n2_b0, p), _pad2(n2_w1, p, p), _row(n2_b1, p),
                    _row(n2_g, p), _row(n2_be, p)],
                   [hid])

    dblk = bm["blocks"][0]
    de_w0, de_b0, de_w1, de_b1, de_g, de_be = _mlp_parts(dblk["edge"])

    sw0, sb0, sw1, sb1, sg, sbe = _mlp_parts(bm["src_emb"])

    def _src_body(xs_, ws_):
        (xx,) = xs_
        w0_, b0_, w1_, b1_, g_, be_, pa_ = ws_
        y = _ln(_mm(_silu(_mm(xx, w0_) + b0_), w1_) + b1_, g_, be_, hid)
        return (_mm(y, pa_),)

    (a3,) = _rows(_src_body, [xlp],
                  [sw0, _row(sb0), sw1, _row(sb1), _row(sg), _row(sbe),
                   de_w0[:hid]], [hid])

    xd2, b3 = _emb_proj(xs, bm["dst_emb"], [de_w0[hid:2 * hid]], hid)

    s3 = _gather2sum(a3, b3, h2e_s, h2e_d)
    (msg3,) = _msg_stage(h2e_a, s3, bm["edge_emb"], de_w0[2 * hid:], de_b0,
                         de_w1, de_b1, de_g, de_be, hid, hid)
    agg3 = _scatter_featsplit(msg3, h2e_d, era)

    dn_w0, dn_b0, dn_w1, dn_b1, dn_g, dn_be = _mlp_parts(dblk["node"])
    (wo0, bo0), (wo1, bo1) = bm["out"]["layers"]
    inf = wo1.shape[1]

    def _out_body(xs_, ws_):
        xx, agg = xs_
        wa_, wb_, b0_, w1_, b1_, g_, be_, wo0_, bo0_, wo1_, bo1_ = ws_
        hh = _silu(_mm(xx, wa_) + _mm(agg, wb_) + b0_)
        y = xx + _ln(_mm(hh, w1_) + b1_, g_, be_, hid)
        ho = _silu(_mm(y, wo0_) + bo0_)
        return (_mm(ho, wo1_) + bo1_,)

    (out,) = _rows(_out_body, [xd2, agg3],
                   [dn_w0[:hid], dn_w0[hid:], _row(dn_b0), dn_w1, _row(dn_b1),
                    _row(dn_g), _row(dn_be), wo0, _row(bo0), wo1, _row(bo1)],
                   [inf])

    return out.reshape(bs, ens, n, inf)
